# baseline (speedup 1.0000x reference)
import jax
import jax.numpy as jnp
from jax.experimental import pallas as pl


def _gru(x, h, W_ih, W_hh, b_ih, b_hh):
    gi = x @ W_ih.T + b_ih
    gh = h @ W_hh.T + b_hh
    i_r, i_z, i_n = jnp.split(gi, 3, axis=-1)
    h_r, h_z, h_n = jnp.split(gh, 3, axis=-1)
    r = jax.nn.sigmoid(i_r + h_r)
    z = jax.nn.sigmoid(i_z + h_z)
    n = jnp.tanh(i_n + r * h_n)
    return (1.0 - z) * n + z * h


def _seg_softmax(alpha, index, num_segments):
    amax = jax.ops.segment_max(alpha, index, num_segments=num_segments)
    amax = jnp.where(jnp.isfinite(amax), amax, 0.0)
    e = jnp.exp(alpha - amax[index])
    s = jax.ops.segment_sum(e, index, num_segments=num_segments)
    return e / (s[index] + 1e-16)


def _tail_kernel(g_ref, w_ref, b_ref, o_ref):
    o_ref[...] = g_ref[...] @ w_ref[...].T + b_ref[...]


def kernel(node_attr, edge_index, edge_attr, lin1_W, lin1_b, gate_lin1_W, gate_lin2_W, gate_att_l, gate_att_r, gate_bias, gru_W_ih, gru_W_hh, gru_b_ih, gru_b_hh, mol_W, mol_att_src, mol_att_dst, mol_bias, mgru_W_ih, mgru_W_hh, mgru_b_ih, mgru_b_hh, lin2_W, lin2_b):
    src = edge_index[0]
    dst = edge_index[1]
    n = node_attr.shape[0]
    x = jax.nn.leaky_relu(node_attr @ lin1_W.T + lin1_b, 0.01)
    x_j = x[src]
    x_i = x[dst]
    m = jax.nn.leaky_relu(jnp.concatenate([x_j, edge_attr], axis=-1) @ gate_lin1_W.T, 0.01)
    alpha = jax.nn.leaky_relu(m @ gate_att_l + x_i @ gate_att_r, 0.01)
    alpha = _seg_softmax(alpha, dst, n)
    msg = (m @ gate_lin2_W.T) * alpha[:, None]
    h = jax.ops.segment_sum(msg, dst, num_segments=n) + gate_bias
    h = jax.nn.elu(h)
    x = jax.nn.relu(_gru(h, x, gru_W_ih, gru_W_hh, gru_b_ih, gru_b_hh))
    out = jax.nn.relu(jnp.sum(x, axis=0, keepdims=True))
    xs = x @ mol_W.T
    xd = out @ mol_W.T
    a = jax.nn.leaky_relu(xs @ mol_att_src + (xd @ mol_att_dst)[0], 0.01)
    a = jax.nn.softmax(a)
    g = jax.nn.elu(jnp.sum(xs * a[:, None], axis=0, keepdims=True) + mol_bias)
    out = jax.nn.relu(_gru(g, out, mgru_W_ih, mgru_W_hh, mgru_b_ih, mgru_b_hh))
    return pl.pallas_call(
        _tail_kernel,
        out_shape=jax.ShapeDtypeStruct((1, 64), jnp.float32),
    )(out, lin2_W, lin2_b)


# speedup vs baseline: 9.5118x; 9.5118x over previous
import jax
import jax.numpy as jnp
from jax import lax
from jax.experimental import pallas as pl
from jax.experimental.pallas import tpu as pltpu
from jax.experimental.pallas import tpu_sc as plsc

N = 100000
E = 1600000
H = 64
BN = 2048
BE = 2048
EP = 1638400
GEP = EP // BE
PADE = EP - E
NC, NS = 2, 16
NW = NC * NS
EPW = EP // NW
KA = 2048
EPT = EP // NS
KC = 1024
NPAD = 100352
SPT = NPAD // NS
GNP = NPAD // BN
ZW = 1568
ZR = 1568

_F32 = jnp.float32


def _lrelu(v):
    return jnp.maximum(v, 0.01 * v)


def _sigm(v):
    return 1.0 / (1.0 + jnp.exp(-v))


def _elu(v):
    return jnp.where(v > 0, v, jnp.exp(jnp.minimum(v, 0.0)) - 1.0)


def _xT(na_ref, w1, b1):
    return _lrelu(w1[...] * na_ref[0, 0][None, :] + b1[...])


def _gru_new(h, xT, Wih, Whh, bih, bhh):
    gi = jnp.dot(Wih[...], h, preferred_element_type=_F32) + bih[...]
    gh = jnp.dot(Whh[...], xT, preferred_element_type=_F32) + bhh[...]
    rg = _sigm(gi[0:H] + gh[0:H])
    zg = _sigm(gi[H:2 * H] + gh[H:2 * H])
    ng = jnp.tanh(gi[2 * H:3 * H] + rg * gh[2 * H:3 * H])
    return (1.0 - zg) * ng + zg * xT


def _xnew(hn0, hn1, hn2, hn3, ssum, na, W2, gb, Wih, Whh, bih, bhh, w1, b1):
    hT = jnp.concatenate(
        [hn0[...], hn1[...], hn2[...], hn3[...]], axis=0)
    hpre = hT / (ssum[0, 0][None, :] + 1e-16)
    h = _elu(jnp.dot(W2[...], hpre, preferred_element_type=_F32) + gb[...])
    xT = _xT(na, w1, b1)
    return jnp.maximum(_gru_new(h, xT, Wih, Whh, bih, bhh), 0.0)


def _pre_body(na, w1, b1, attr, u_out):
    xT = _xT(na, w1, b1)
    u_out[...] = jnp.sum(xT * attr[...], axis=0).reshape(1, 1, BN)


def _gather_body(na, u, src, dst, asrc, udst, idx_v, val_v, sem):
    c = lax.axis_index("c")
    s = lax.axis_index("s")
    wid = s * NC + c
    base = wid * EPW

    def chunk(i, carry):
        off = base + i * KA
        pltpu.sync_copy(src.at[pl.ds(off, KA)], idx_v)
        pltpu.async_copy(na.at[idx_v], val_v, sem).wait()
        pltpu.sync_copy(val_v, asrc.at[pl.ds(off, KA)])
        pltpu.sync_copy(dst.at[pl.ds(off, KA)], idx_v)
        pltpu.async_copy(u.at[idx_v], val_v, sem).wait()
        pltpu.sync_copy(val_v, udst.at[pl.ds(off, KA)])
        return carry

    lax.fori_loop(0, EPW // KA, chunk, 0)


def _edge_body(asrc, udst, t, w1, b1, W1a, we, attl,
               e_out, m0, m1, m2, m3):
    xT = _xT(asrc, w1, b1)
    cT = jnp.dot(W1a[...], xT, preferred_element_type=_F32)
    mT = _lrelu(cT + we[...] * t[0, 0][None, :])
    sv = jnp.sum(mT * attl[...], axis=0)
    alpha = _lrelu(sv + udst[0, 0])
    ev = jnp.exp(alpha)
    e_out[...] = ev.reshape(1, 1, BE)
    msgT = mT * ev[None, :]
    outs = (m0, m1, m2, m3)
    for g in range(4):
        outs[g][...] = msgT[g * 16:(g + 1) * 16, :]


def _scatter_body(msg0, msg1, msg2, msg3, e, dst,
                  hn0, hn1, hn2, hn3, ss,
                  acc, sacc, data_v, idx_v, idx2_v, e_v, zbuf, sem):
    del sem
    c = lax.axis_index("c")
    s = lax.axis_index("s")
    msgs = (msg0, msg1, msg2, msg3)
    hns = (hn0, hn1, hn2, hn3)
    zeros16 = jnp.zeros((16,), _F32)

    def zrow(i, carry):
        zbuf[pl.ds(i * 16, 16)] = zeros16
        return carry

    lax.fori_loop(0, ZW // 16, zrow, 0)

    APT = 16 * NPAD // NS

    for core in range(NC):
        @pl.when(c == core)
        def _(core=core):
            for r in range(2):
                g = core * 2 + r

                def zacc(j, carry):
                    pltpu.sync_copy(zbuf, acc.at[pl.ds(s * APT + j * ZW, ZW)])
                    return carry

                lax.fori_loop(0, APT // ZW, zacc, 0)
                if g == 0:
                    def zsacc(j, carry):
                        pltpu.sync_copy(
                            zbuf, sacc.at[pl.ds(s * SPT + j * ZW, ZW)])
                        return carry
                    lax.fori_loop(0, SPT // ZW, zsacc, 0)
                plsc.subcore_barrier()

                def chunk(i, carry, g=g):
                    eb = s * EPT + i * KC
                    pltpu.sync_copy(dst.at[pl.ds(eb, KC)], idx_v)
                    pltpu.sync_copy(msgs[g].at[:, pl.ds(eb, KC)], data_v)
                    if g == 0:
                        pltpu.sync_copy(e.at[pl.ds(eb, KC)], e_v)
                        pltpu.sync_copy(e_v, sacc.at[idx_v], add=True)
                    for ch in range(16):
                        def addoff(j, carry, ch=ch):
                            idx2_v[pl.ds(j * 16, 16)] = (
                                idx_v[pl.ds(j * 16, 16)] + ch * NPAD)
                            return carry
                        lax.fori_loop(0, KC // 16, addoff, 0)
                        pltpu.sync_copy(data_v.at[ch], acc.at[idx2_v],
                                        add=True)
                    return carry

                lax.fori_loop(0, EPT // KC, chunk, 0)
                plsc.subcore_barrier()
                pltpu.sync_copy(acc.at[pl.ds(s * APT, APT)],
                                hns[g].at[pl.ds(s * APT, APT)])
                if g == 0:
                    pltpu.sync_copy(sacc.at[pl.ds(s * SPT, SPT)],
                                    ss.at[pl.ds(s * SPT, SPT)])
                plsc.subcore_barrier()


def _mask2d(i):
    return (lax.broadcasted_iota(jnp.int32, (1, BN), 1) + i * BN) < N


def _nodesum_body(hn0, hn1, hn2, hn3, ssum, na, W2, gb, Wih, Whh, bih, bhh,
                  w1, b1, sumx):
    i = pl.program_id(0)
    xn = _xnew(hn0, hn1, hn2, hn3, ssum, na, W2, gb, Wih, Whh, bih, bhh,
               w1, b1)
    xn = jnp.where(_mask2d(i), xn, 0.0)

    @pl.when(i == 0)
    def _():
        sumx[...] = jnp.zeros_like(sumx)

    sumx[...] += jnp.sum(xn, axis=1, keepdims=True)


def _readout_body(hn0, hn1, hn2, hn3, ssum, na, W2, gb, Wih, Whh, bih, bhh,
                  w1, b1, sumx, molW, atts, attd, num, den):
    i = pl.program_id(0)
    xn = _xnew(hn0, hn1, hn2, hn3, ssum, na, W2, gb, Wih, Whh, bih, bhh,
               w1, b1)
    xs = jnp.dot(molW[...], xn, preferred_element_type=_F32)
    out0 = jnp.maximum(sumx[...], 0.0)
    xd = jnp.dot(molW[...], out0, preferred_element_type=_F32)
    const = jnp.sum(xd * attd[...])
    ap = _lrelu(jnp.sum(xs * atts[...], axis=0) + const)
    w = jnp.where(_mask2d(i), jnp.exp(ap)[None, :], 0.0)

    @pl.when(i == 0)
    def _():
        num[...] = jnp.zeros_like(num)
        den[...] = jnp.zeros_like(den)

    num[...] += jnp.sum(xs * w, axis=1, keepdims=True)
    den[...] += jnp.sum(w).reshape(1, 1)


def _final_body(sumx, num, den, molb, mWih, mWhh, mbih, mbhh, l2W, l2b, out):
    out0 = jnp.maximum(sumx[...], 0.0)
    g = _elu(num[...] / den[0, 0] + molb[...])
    o = jnp.maximum(_gru_new(g, out0, mWih, mWhh, mbih, mbhh), 0.0)
    out[...] = jnp.dot(l2W[...], o, preferred_element_type=_F32) + l2b[...]


def _v3n(x):
    return x.reshape(GNP, 1, BN)


def _v3e(x):
    return x.reshape(GEP, 1, BE)


_N3 = pl.BlockSpec((1, 1, BN), lambda i: (i, 0, 0))
_E3 = pl.BlockSpec((1, 1, BE), lambda i: (i, 0, 0))


def _wspec(shape):
    return pl.BlockSpec(shape, lambda *_: tuple(0 for _ in shape))


def kernel(node_attr, edge_index, edge_attr, lin1_W, lin1_b, gate_lin1_W,
           gate_lin2_W, gate_att_l, gate_att_r, gate_bias, gru_W_ih, gru_W_hh,
           gru_b_ih, gru_b_hh, mol_W, mol_att_src, mol_att_dst, mol_bias,
           mgru_W_ih, mgru_W_hh, mgru_b_ih, mgru_b_hh, lin2_W, lin2_b):
    f32 = _F32
    na = jnp.concatenate([node_attr.reshape(N),
                          jnp.zeros((NPAD - N,), jnp.float32)])
    izeros = jnp.zeros((PADE,), jnp.int32)
    src = jnp.concatenate([edge_index[0], izeros])
    dst = jnp.concatenate([edge_index[1],
                           N + (jnp.arange(PADE, dtype=jnp.int32)
                                % (NPAD - N))])
    t_pad = jnp.concatenate([edge_attr.reshape(E),
                             jnp.zeros((PADE,), jnp.float32)])
    b1 = lin1_b.reshape(H, 1)
    W1a = gate_lin1_W[:, :H]
    we = gate_lin1_W[:, H:H + 1]
    attl = gate_att_l.reshape(H, 1)
    attr_ = gate_att_r.reshape(H, 1)
    gb = gate_bias.reshape(H, 1)
    bih = gru_b_ih.reshape(3 * H, 1)
    bhh = gru_b_hh.reshape(3 * H, 1)
    atts = mol_att_src.reshape(H, 1)
    attd = mol_att_dst.reshape(H, 1)
    molb = mol_bias.reshape(H, 1)
    mbih = mgru_b_ih.reshape(3 * H, 1)
    mbhh = mgru_b_hh.reshape(3 * H, 1)
    l2b = lin2_b.reshape(H, 1)

    u3 = pl.pallas_call(
        _pre_body,
        grid=(GNP,),
        in_specs=[_N3, _wspec((H, 1)), _wspec((H, 1)), _wspec((H, 1))],
        out_specs=_N3,
        out_shape=jax.ShapeDtypeStruct((GNP, 1, BN), f32),
    )(_v3n(na), lin1_W, b1, attr_)

    asrc, udst = pl.kernel(
        _gather_body,
        out_type=(jax.ShapeDtypeStruct((EP,), f32),
                  jax.ShapeDtypeStruct((EP,), f32)),
        mesh=plsc.VectorSubcoreMesh(core_axis_name="c", subcore_axis_name="s"),
        scratch_types=(pltpu.VMEM((KA,), jnp.int32),
                       pltpu.VMEM((KA,), f32),
                       pltpu.SemaphoreType.DMA),
    )(na, u3.reshape(NPAD), src, dst)

    e3, m0, m1, m2, m3 = pl.pallas_call(
        _edge_body,
        grid=(GEP,),
        in_specs=[_E3, _E3, _E3,
                  _wspec((H, 1)), _wspec((H, 1)), _wspec((H, H)),
                  _wspec((H, 1)), _wspec((H, 1))],
        out_specs=[_E3] + [pl.BlockSpec((16, BE), lambda i: (0, i))] * 4,
        out_shape=[jax.ShapeDtypeStruct((GEP, 1, BE), f32)] +
                  [jax.ShapeDtypeStruct((16, EP), f32)] * 4,
    )(_v3e(asrc), _v3e(udst), _v3e(t_pad),
      lin1_W, b1, W1a, we, attl)

    hn0, hn1, hn2, hn3, ssum_pad = pl.kernel(
        _scatter_body,
        out_type=tuple([jax.ShapeDtypeStruct((16 * NPAD,), f32)] * 4) +
                 (jax.ShapeDtypeStruct((NPAD,), f32),),
        mesh=plsc.VectorSubcoreMesh(core_axis_name="c", subcore_axis_name="s"),
        compiler_params=pltpu.CompilerParams(use_tc_tiling_on_sc=False),
        scratch_types=(pltpu.VMEM_SHARED((16 * NPAD,), f32),
                       pltpu.VMEM_SHARED((NPAD,), f32),
                       pltpu.VMEM((16, KC), f32),
                       pltpu.VMEM((KC,), jnp.int32),
                       pltpu.VMEM((KC,), jnp.int32),
                       pltpu.VMEM((KC,), f32),
                       pltpu.VMEM((ZW,), f32),
                       pltpu.SemaphoreType.DMA),
    )(m0, m1, m2, m3, e3.reshape(EP), dst)
    ssum3 = _v3n(ssum_pad)
    hn0, hn1, hn2, hn3 = (h.reshape(16, NPAD) for h in (hn0, hn1, hn2, hn3))

    hnspec = [pl.BlockSpec((16, BN), lambda i: (0, i))] * 4
    node_ins = [hn0, hn1, hn2, hn3, ssum3, _v3n(na), gate_lin2_W, gb,
                gru_W_ih, gru_W_hh, bih, bhh, lin1_W, b1]
    node_specs = hnspec + [_N3, _N3, _wspec((H, H)), _wspec((H, 1)),
                           _wspec((3 * H, H)), _wspec((3 * H, H)),
                           _wspec((3 * H, 1)), _wspec((3 * H, 1)),
                           _wspec((H, 1)), _wspec((H, 1))]

    sumx = pl.pallas_call(
        _nodesum_body,
        grid=(GNP,),
        in_specs=node_specs,
        out_specs=pl.BlockSpec((H, 1), lambda i: (0, 0)),
        out_shape=jax.ShapeDtypeStruct((H, 1), f32),
    )(*node_ins)

    num, den = pl.pallas_call(
        _readout_body,
        grid=(GNP,),
        in_specs=node_specs + [_wspec((H, 1)), _wspec((H, H)),
                               _wspec((H, 1)), _wspec((H, 1))],
        out_specs=[pl.BlockSpec((H, 1), lambda i: (0, 0)),
                   pl.BlockSpec((1, 1), lambda i: (0, 0))],
        out_shape=[jax.ShapeDtypeStruct((H, 1), f32),
                   jax.ShapeDtypeStruct((1, 1), f32)],
    )(*node_ins, sumx, mol_W, atts, attd)

    res = pl.pallas_call(
        _final_body,
        in_specs=[_wspec((H, 1)), _wspec((H, 1)), _wspec((1, 1)),
                  _wspec((H, 1)), _wspec((3 * H, H)), _wspec((3 * H, H)),
                  _wspec((3 * H, 1)), _wspec((3 * H, 1)),
                  _wspec((H, H)), _wspec((H, 1))],
        out_specs=_wspec((H, 1)),
        out_shape=jax.ShapeDtypeStruct((H, 1), f32),
    )(sumx, num, den, molb, mgru_W_ih, mgru_W_hh, mbih, mbhh, lin2_W, l2b)
    return res.reshape(1, H)
